# Initial kernel scaffold; baseline (speedup 1.0000x reference)
import jax
import jax.numpy as jnp
from jax.experimental import pallas as pl


def kernel(x, e, edge_index, node_W, node_b, edge_W, edge_b, Wu, bu, Wv, bv, Wa, ba, Wb, bb, Wc, bc, ln_x_g, ln_x_b, ln_e_g, ln_e_b, out_W, out_b):
    raise NotImplementedError("write your pallas kernel here")


# speedup vs baseline: 1.0720x; 1.0720x over previous
import jax
import jax.numpy as jnp
from jax.experimental import pallas as pl
from jax.experimental.pallas import tpu as pltpu

N = 10000
E = 320000
H = 256
L = 12

RN = 1000
RE = 1000


def _ln_relu(t, g, b):
    mu = jnp.mean(t, axis=-1, keepdims=True)
    var = jnp.mean((t - mu) ** 2, axis=-1, keepdims=True)
    return jnp.maximum((t - mu) * jax.lax.rsqrt(var + 1e-5) * g + b, 0.0)


def _node0_body(x_ref, nW_ref, nb_ref, Wc_ref, bc_ref,
                h_ref, u_ref, v_ref, a_ref, b_ref):
    h = x_ref[...] @ nW_ref[...] + nb_ref[...]
    y = h @ Wc_ref[...] + bc_ref[...]
    h_ref[...] = h
    u_ref[...] = y[:, 0 * H:1 * H]
    v_ref[...] = y[:, 1 * H:2 * H]
    a_ref[...] = y[:, 2 * H:3 * H]
    b_ref[...] = y[:, 3 * H:4 * H]


def _node_body(h_ref, u_in, agg_ref, g_ref, be_ref, Wc_ref, bc_ref,
               h_ref_o, u_ref, v_ref, a_ref, b_ref):
    hn = h_ref[...] + _ln_relu(u_in[...] + agg_ref[...], g_ref[...], be_ref[...])
    y = hn @ Wc_ref[...] + bc_ref[...]
    h_ref_o[...] = hn
    u_ref[...] = y[:, 0 * H:1 * H]
    v_ref[...] = y[:, 1 * H:2 * H]
    a_ref[...] = y[:, 2 * H:3 * H]
    b_ref[...] = y[:, 3 * H:4 * H]


def _node_last_body(h_ref, u_in, agg_ref, g_ref, be_ref, h_ref_o):
    h_ref_o[...] = h_ref[...] + _ln_relu(u_in[...] + agg_ref[...],
                                         g_ref[...], be_ref[...])


def _edge0_body(e_ref, eW_ref, eb_ref, Wc_ref, bc_ref, f_ref, ce_ref):
    f = e_ref[...] @ eW_ref[...] + eb_ref[...]
    ce_ref[...] = f @ Wc_ref[...] + bc_ref[...]
    f_ref[...] = f


def _edge_body(f_ref, en_ref, g_ref, be_ref, Wc_ref, bc_ref, f_o, ce_ref):
    fn = f_ref[...] + _ln_relu(en_ref[...], g_ref[...], be_ref[...])
    ce_ref[...] = fn @ Wc_ref[...] + bc_ref[...]
    f_o[...] = fn


def _edge_last_body(f_ref, en_ref, g_ref, be_ref, oW_ref, ob_ref, out_ref):
    fn = f_ref[...] + _ln_relu(en_ref[...], g_ref[...], be_ref[...])
    out_ref[...] = fn @ oW_ref[...] + ob_ref[...]


def _rows(bs):
    return pl.BlockSpec(bs, lambda r: (r, 0))


def _full(shape):
    return pl.BlockSpec(shape, lambda r: tuple(0 for _ in shape))


_f32 = jnp.float32


def _node0(x, nW, nb, Wcat, bcat):
    return pl.pallas_call(
        _node0_body,
        grid=(N // RN,),
        in_specs=[_rows((RN, 2)), _full((2, H)), _full((1, H)),
                  _full((H, 4 * H)), _full((1, 4 * H))],
        out_specs=[_rows((RN, H))] * 5,
        out_shape=[jax.ShapeDtypeStruct((N, H), _f32)] * 5,
    )(x, nW, nb, Wcat, bcat)


def _node(h, u, agg, g, be, Wcat, bcat):
    return pl.pallas_call(
        _node_body,
        grid=(N // RN,),
        in_specs=[_rows((RN, H))] * 3 + [_full((1, H))] * 2 +
                 [_full((H, 4 * H)), _full((1, 4 * H))],
        out_specs=[_rows((RN, H))] * 5,
        out_shape=[jax.ShapeDtypeStruct((N, H), _f32)] * 5,
    )(h, u, agg, g, be, Wcat, bcat)


def _node_last(h, u, agg, g, be):
    return pl.pallas_call(
        _node_last_body,
        grid=(N // RN,),
        in_specs=[_rows((RN, H))] * 3 + [_full((1, H))] * 2,
        out_specs=[_rows((RN, H))],
        out_shape=[jax.ShapeDtypeStruct((N, H), _f32)],
    )(h, u, agg, g, be)[0]


def _edge0(e2, eW, eb, Wc, bc):
    return pl.pallas_call(
        _edge0_body,
        grid=(E // RE,),
        in_specs=[_rows((RE, 1)), _full((1, H)), _full((1, H)),
                  _full((H, H)), _full((1, H))],
        out_specs=[_rows((RE, H))] * 2,
        out_shape=[jax.ShapeDtypeStruct((E, H), _f32)] * 2,
    )(e2, eW, eb, Wc, bc)


def _edge(f, en, g, be, Wc, bc):
    return pl.pallas_call(
        _edge_body,
        grid=(E // RE,),
        in_specs=[_rows((RE, H))] * 2 + [_full((1, H))] * 2 +
                 [_full((H, H)), _full((1, H))],
        out_specs=[_rows((RE, H))] * 2,
        out_shape=[jax.ShapeDtypeStruct((E, H), _f32)] * 2,
    )(f, en, g, be, Wc, bc)


def _edge_last(f, en, g, be, oW, ob):
    return pl.pallas_call(
        _edge_last_body,
        grid=(E // RE,),
        in_specs=[_rows((RE, H))] * 2 + [_full((1, H))] * 2 +
                 [_full((H, 2)), _full((1, 2))],
        out_specs=[_rows((RE, 2))],
        out_shape=[jax.ShapeDtypeStruct((E, 2), _f32)],
    )(f, en, g, be, oW, ob)[0]


def _edge_stage(ax, bx, vx, ce, src, dst):
    e_new = ax[src] + bx[dst] + ce
    gates = jax.nn.sigmoid(e_new)
    agg = jnp.zeros((N, H), _f32).at[dst].add(gates * vx[src])
    return e_new, agg


def kernel(x, e, edge_index, node_W, node_b, edge_W, edge_b,
           Wu, bu, Wv, bv, Wa, ba, Wb, bb, Wc, bc,
           ln_x_g, ln_x_b, ln_e_g, ln_e_b, out_W, out_b):
    src = edge_index[0]
    dst = edge_index[1]
    Wcat = jnp.concatenate([Wu, Wv, Wa, Wb], axis=2)
    bcat = jnp.concatenate([bu, bv, ba, bb], axis=1)

    r1 = lambda v: v.reshape(1, -1)

    h, ux, vx, ax, bx = _node0(x, node_W, r1(node_b), Wcat[0], r1(bcat[0]))
    f, ce = _edge0(e.reshape(E, 1), edge_W, r1(edge_b), Wc[0], r1(bc[0]))

    for i in range(L):
        e_new, agg = _edge_stage(ax, bx, vx, ce, src, dst)
        gx, bxx = r1(ln_x_g[i]), r1(ln_x_b[i])
        ge, bee = r1(ln_e_g[i]), r1(ln_e_b[i])
        if i < L - 1:
            h, ux, vx, ax, bx = _node(h, ux, agg, gx, bxx,
                                      Wcat[i + 1], r1(bcat[i + 1]))
            f, ce = _edge(f, e_new, ge, bee, Wc[i + 1], r1(bc[i + 1]))
        else:
            h = _node_last(h, ux, agg, gx, bxx)
            e_out = _edge_last(f, e_new, ge, bee, out_W, r1(out_b))
    return (h, e_out)
